# baseline (speedup 1.0000x reference)
import dataclasses

import jax
import jax.numpy as jnp
from jax import lax
from jax.experimental import pallas as pl
from jax.experimental.pallas import tpu as pltpu
from jax.experimental.pallas import tpu_sc as plsc

_N = 10000
_E = 320000
_DIN = 128
_H = 64
_G = 64
_NT = 32
_TPB = 320
_NPAD = _NT * _TPB
_E2 = _E + _N
_E2P = 330240
_SCK = 2560
_CK = 256
_CAPT = 24576
_CAPB = _CAPT + _SCK

_MESH = plsc.VectorSubcoreMesh(core_axis_name="c", subcore_axis_name="s")

_SC_PARAMS = pltpu.CompilerParams()
if "needs_layout_passes" in pltpu.CompilerParams.__dataclass_fields__:
    _SC_PARAMS = dataclasses.replace(_SC_PARAMS, needs_layout_passes=False)


def _wid():
    return lax.axis_index("s") * 2 + lax.axis_index("c")


def _prep_body(src_hbm, dst_hbm, lsrc_hbm, ldst_hbm, cnt_hbm,
               src_v, dst_v, stage_s, stage_d, cnt_v, sem_a, sem_b, sem):
    wid = _wid()
    lo = wid * _TPB

    zero16 = jnp.zeros((16,), jnp.int32)

    dump16 = jnp.full((16,), _TPB, jnp.int32)

    @pl.loop(0, _CAPB, step=16)
    def _(i):
        stage_s[pl.ds(i, 16)] = zero16
        stage_d[pl.ds(i, 16)] = dump16

    nchunks = _E2P // _SCK

    def issue(g, b):
        gc = jnp.minimum(g, nchunks - 1)
        s = sem_a if b == 0 else sem_b
        pltpu.async_copy(src_hbm.at[pl.ds(gc * _SCK, _SCK)], src_v.at[b], s)
        pltpu.async_copy(dst_hbm.at[pl.ds(gc * _SCK, _SCK)], dst_v.at[b], s)

    def wait(b):
        s = sem_a if b == 0 else sem_b
        pltpu.make_async_copy(src_hbm.at[pl.ds(0, _SCK)], src_v.at[b],
                              s).wait()
        pltpu.make_async_copy(dst_hbm.at[pl.ds(0, _SCK)], dst_v.at[b],
                              s).wait()

    def scan(b, cnt):
        def grp(i, cnt):
            s16 = src_v[b, pl.ds(i * 16, 16)]
            d16 = dst_v[b, pl.ds(i * 16, 16)]
            dl = d16 - lo
            m = (dl >= 0) & (dl < _TPB)
            plsc.store_compressed(stage_s.at[pl.ds(cnt, 16)], s16, mask=m)
            plsc.store_compressed(stage_d.at[pl.ds(cnt, 16)], dl, mask=m)
            pc = plsc.all_reduce_population_count(m)
            return cnt + lax.squeeze(lax.slice(pc, (0,), (1,)), (0,))

        return lax.cond(cnt <= _CAPT,
                        lambda c: lax.fori_loop(0, _SCK // 16, grp, c),
                        lambda c: c, cnt)

    issue(0, 0)
    issue(1, 1)

    def pair(p, cnt):
        g0 = 2 * p
        wait(0)
        cnt = scan(0, cnt)
        issue(g0 + 2, 0)
        wait(1)
        cnt = scan(1, cnt)
        issue(g0 + 3, 1)
        return cnt

    cnt = lax.fori_loop(0, (nchunks - 1) // 2, pair, jnp.int32(0))
    wait(0)
    cnt = scan(0, cnt)
    wait(1)
    cnt = jnp.minimum(cnt, jnp.int32(_CAPT))
    cnt_v[...] = jnp.full((16,), cnt, jnp.int32)
    pltpu.async_copy(cnt_v, cnt_hbm.at[pl.ds(wid * 16, 16)], sem).wait()
    pltpu.async_copy(stage_s.at[pl.ds(0, _CAPT)],
                     lsrc_hbm.at[pl.ds(wid * _CAPT, _CAPT)], sem).wait()
    pltpu.async_copy(stage_d.at[pl.ds(0, _CAPT)],
                     ldst_hbm.at[pl.ds(wid * _CAPT, _CAPT)], sem).wait()


def _prep(src, dst):
    return pl.kernel(
        _prep_body,
        out_type=(
            jax.ShapeDtypeStruct((_NT * _CAPT,), jnp.int32),
            jax.ShapeDtypeStruct((_NT * _CAPT,), jnp.int32),
            jax.ShapeDtypeStruct((_NT * 16,), jnp.int32),
        ),
        mesh=_MESH,
        scratch_types=[
            pltpu.VMEM((2, _SCK), jnp.int32),
            pltpu.VMEM((2, _SCK), jnp.int32),
            pltpu.VMEM((_CAPB,), jnp.int32),
            pltpu.VMEM((_CAPB,), jnp.int32),
            pltpu.VMEM((16,), jnp.int32),
            pltpu.SemaphoreType.DMA,
            pltpu.SemaphoreType.DMA,
            pltpu.SemaphoreType.DMA,
        ],
        compiler_params=_SC_PARAMS,
    )(src, dst)


def _edge_body(xlc_hbm, xrc_hbm, att_hbm, lsrc_hbm, ldst_hbm, cnt_hbm,
               outn_hbm, outs_hbm, xr_loc, att_v, cnt_v, idx_s0, idx_s1,
               idx_d0, idx_d1, rows, acc_n, acc_s,
               semi0, semi1, semg0, semg1, semm):
    wid = _wid()
    lo = wid * _TPB

    pltpu.async_copy(cnt_hbm.at[pl.ds(wid * 16, 16)], cnt_v, semm).wait()
    pltpu.async_copy(att_hbm, att_v, semm).wait()
    xr_cp = pltpu.async_copy(xrc_hbm.at[pl.ds(lo, _TPB)], xr_loc, semm)

    zero16 = jnp.zeros((16,), jnp.float32)

    @pl.loop(0, _TPB + 1)
    def _(r):
        for c in range(8):
            acc_n[r, pl.ds(c * 16, 16)] = zero16

    @pl.loop(0, 32 * (_TPB + 1), step=16)
    def _(i):
        acc_s[pl.ds(i, 16)] = zero16

    xr_cp.wait()

    cnt = cnt_v[...][0]
    a = [att_v[pl.ds(c * 16, 16)] for c in range(8)]
    lane0f = jnp.where(lax.iota(jnp.int32, 16) == 0, 1.0, 0.0)

    nch = (cnt + _CK - 1) // _CK
    gmax = _CAPT // _CK - 1

    def issue_idx(g, b):
        base = wid * _CAPT + jnp.minimum(g, gmax) * _CK
        s = semi0 if b == 0 else semi1
        i_s = idx_s0 if b == 0 else idx_s1
        i_d = idx_d0 if b == 0 else idx_d1
        pltpu.async_copy(lsrc_hbm.at[pl.ds(base, _CK)], i_s, s)
        pltpu.async_copy(ldst_hbm.at[pl.ds(base, _CK)], i_d, s)

    def wait_idx(b):
        s = semi0 if b == 0 else semi1
        i_s = idx_s0 if b == 0 else idx_s1
        i_d = idx_d0 if b == 0 else idx_d1
        pltpu.make_async_copy(lsrc_hbm.at[pl.ds(0, _CK)], i_s, s).wait()
        pltpu.make_async_copy(ldst_hbm.at[pl.ds(0, _CK)], i_d, s).wait()

    def issue_gather(b):
        i_s = idx_s0 if b == 0 else idx_s1
        pltpu.async_copy(xlc_hbm.at[i_s], rows, semg0)

    def wait_gather(b):
        i_s = idx_s0 if b == 0 else idx_s1
        pltpu.make_async_copy(xlc_hbm.at[i_s], rows, semg0).wait()

    last = jnp.full((16,), 15, jnp.int32)

    def compute(g, b):
        i_d = idx_d0 if b == 0 else idx_d1

        def grp(g16, _):
            dl16 = i_d[pl.ds(g16 * 16, 16)]
            for j in range(16):
                e = g16 * 16 + j
                dl = dl16[j]
                xm = [rows[e, pl.ds(c * 16, 16)] for c in range(4)]
                xv = [rows[e, pl.ds(64 + c * 16, 16)] for c in range(4)]
                pm = pv = None
                for c in range(4):
                    tm = xm[c] + xr_loc[dl, pl.ds(c * 16, 16)]
                    tm = jnp.maximum(tm, 0.2 * tm) * a[c]
                    pm = tm if pm is None else pm + tm
                    tv = xv[c] + xr_loc[dl, pl.ds(64 + c * 16, 16)]
                    tv = jnp.maximum(tv, 0.2 * tv) * a[4 + c]
                    pv = tv if pv is None else pv + tv
                evm = jnp.exp(plsc.cumsum(pm).at[last]
                              .get(mode="promise_in_bounds"))
                evv = jnp.exp(plsc.cumsum(pv).at[last]
                              .get(mode="promise_in_bounds"))
                for c in range(4):
                    plsc.addupdate(acc_n.at[dl, pl.ds(c * 16, 16)],
                                   evm * xm[c])
                    plsc.addupdate(acc_n.at[dl, pl.ds(64 + c * 16, 16)],
                                   evv * xv[c])
                plsc.addupdate(acc_s.at[pl.ds(dl * 32, 16)], evm * lane0f)
                plsc.addupdate(acc_s.at[pl.ds(dl * 32 + 16, 16)],
                               evv * lane0f)
            return 0

        lax.fori_loop(0, _CK // 16, grp, 0)

    issue_idx(0, 0)

    def chunk(p, _):
        g0 = 2 * p
        wait_idx(0)
        issue_gather(0)
        issue_idx(g0 + 1, 1)
        wait_gather(0)
        compute(g0, 0)
        wait_idx(1)
        issue_gather(1)
        issue_idx(g0 + 2, 0)
        wait_gather(1)
        compute(g0 + 1, 1)
        return 0

    lax.fori_loop(0, (nch + 1) // 2, chunk, 0)
    wait_idx(0)

    pltpu.async_copy(acc_n.at[pl.ds(0, _TPB)], outn_hbm.at[wid],
                     semm).wait()
    pltpu.async_copy(acc_s.at[pl.ds(0, 32 * _TPB)],
                     outs_hbm.at[pl.ds(wid * 32 * _TPB, 32 * _TPB)],
                     semm).wait()


def _edge_call(xlc, xrc, att, lsrc, ldst, cnts):
    return pl.kernel(
        _edge_body,
        out_type=(
            jax.ShapeDtypeStruct((_NT, _TPB, 2 * _H), jnp.float32),
            jax.ShapeDtypeStruct((_NT * _TPB * 32,), jnp.float32),
        ),
        mesh=_MESH,
        scratch_types=[
            pltpu.VMEM((_TPB, 2 * _H), jnp.float32),
            pltpu.VMEM((2 * _H,), jnp.float32),
            pltpu.VMEM((16,), jnp.int32),
            pltpu.VMEM((_CK,), jnp.int32),
            pltpu.VMEM((_CK,), jnp.int32),
            pltpu.VMEM((_CK,), jnp.int32),
            pltpu.VMEM((_CK,), jnp.int32),
            pltpu.VMEM((_CK, 2 * _H), jnp.float32),
            pltpu.VMEM((_TPB + 1, 2 * _H), jnp.float32),
            pltpu.VMEM((32 * (_TPB + 1),), jnp.float32),
            pltpu.SemaphoreType.DMA,
            pltpu.SemaphoreType.DMA,
            pltpu.SemaphoreType.DMA,
            pltpu.SemaphoreType.DMA,
            pltpu.SemaphoreType.DMA,
        ],
        compiler_params=_SC_PARAMS,
    )(xlc, xrc, att, lsrc, ldst, cnts)


def _elu(h):
    return jnp.where(h > 0, h, jnp.exp(h) - 1.0)


def _finish(num, s, b):
    return _elu(num / (s + 1e-16) + b)


def _proj_first_body(x_ref, wl_ref, wr_ref, bl_ref, br_ref, xlc, xrc):
    x = x_ref[...]
    xlc[...] = jnp.dot(x, wl_ref[...],
                       preferred_element_type=jnp.float32) + bl_ref[...]
    xrc[...] = jnp.dot(x, wr_ref[...],
                       preferred_element_type=jnp.float32) + br_ref[...]


def _mid_body(nums_ref, sv_ref, bpm_ref, bpv_ref, wl_ref, wr_ref,
              bl_ref, br_ref, xlc, xrc):
    nums = nums_ref[...]
    sv = sv_ref[...]
    h = jnp.concatenate(
        [_finish(nums[:, :_H], sv[:, 0:1], bpm_ref[...]),
         _finish(nums[:, _H:], sv[:, 16:17], bpv_ref[...])], axis=1)
    xlc[...] = jnp.dot(h, wl_ref[...],
                       preferred_element_type=jnp.float32) + bl_ref[...]
    xrc[...] = jnp.dot(h, wr_ref[...],
                       preferred_element_type=jnp.float32) + br_ref[...]


def _head_body(nums_ref, sv_ref, bpm_ref, bpv_ref, eps_ref, batch_ref,
               w1_ref, b1_ref, w2_ref, b2_ref, logp, mean_o, logv_o):
    nums = nums_ref[...]
    sv = sv_ref[...]
    mean = _finish(nums[:, :_H], sv[:, 0:1], bpm_ref[...])
    logv = _finish(nums[:, _H:], sv[:, 16:17], bpv_ref[...])
    mean_o[...] = mean
    logv_o[...] = logv
    z = mean + eps_ref[...] * jnp.exp(0.5 * logv)
    gi = lax.broadcasted_iota(jnp.int32, (_G, _NPAD), 0)
    onehot = jnp.where(batch_ref[...] == gi, 1.0, 0.0)
    sums = jnp.dot(onehot, z, preferred_element_type=jnp.float32)
    cnt = jnp.sum(onehot, axis=1, keepdims=True)
    zg = sums / jnp.maximum(cnt, 1.0)
    h = jnp.maximum(
        jnp.dot(zg, w1_ref[...], preferred_element_type=jnp.float32)
        + b1_ref[...], 0.0)
    out = jnp.dot(h, w2_ref[...], preferred_element_type=jnp.float32) \
        + b2_ref[...]
    logits = out[:, :2]
    m = jnp.max(logits, axis=1, keepdims=True)
    lse = m + jnp.log(jnp.sum(jnp.exp(logits - m), axis=1, keepdims=True))
    logp[...] = logits - lse


def _tc_call(body, out_type, *args):
    return pl.pallas_call(body, out_shape=out_type)(*args)


_XO = jax.ShapeDtypeStruct((_NPAD, 2 * _H), jnp.float32)
_HO = jax.ShapeDtypeStruct((_NPAD, _H), jnp.float32)


def _blockdiag(wm, wv):
    z = jnp.zeros((2 * _H, 2 * _H), jnp.float32)
    return z.at[:_H, :_H].set(wm).at[_H:, _H:].set(wv)


def kernel(x, edge_index, batch, mean_params, var_params, fc_params):
    ar = jnp.arange(_N, dtype=jnp.int32)
    npad_e = _E2P - _E2
    src = jnp.concatenate([edge_index[0].astype(jnp.int32), ar,
                           jnp.zeros((npad_e,), jnp.int32)])
    dst = jnp.concatenate([edge_index[1].astype(jnp.int32), ar,
                           jnp.full((npad_e,), 1 << 30, jnp.int32)])
    lsrc, ldst, cnts = _prep(src, dst)

    x_pad = jnp.concatenate([x, jnp.zeros((_NPAD - _N, _DIN), x.dtype)])

    Wl, Wr, Bl, Br, Att, Bo_m, Bo_v = [], [], [], [], [], [], []
    for i in range(3):
        wlm, blm, wrm, brm, attm, bm = mean_params[i]
        wlv, blv, wrv, brv, attv, bv = var_params[i]
        if i == 0:
            Wl.append(jnp.concatenate([wlm, wlv], axis=1))
            Wr.append(jnp.concatenate([wrm, wrv], axis=1))
        else:
            Wl.append(_blockdiag(wlm, wlv))
            Wr.append(_blockdiag(wrm, wrv))
        Bl.append(jnp.concatenate([blm, blv]).reshape(1, 2 * _H))
        Br.append(jnp.concatenate([brm, brv]).reshape(1, 2 * _H))
        Att.append(jnp.concatenate([attm, attv]))
        Bo_m.append(bm.reshape(1, _H))
        Bo_v.append(bv.reshape(1, _H))

    xlc, xrc = _tc_call(_proj_first_body, (_XO, _XO),
                        x_pad, Wl[0], Wr[0], Bl[0], Br[0])

    for i in range(3):
        out_n, out_s = _edge_call(xlc, xrc, Att[i], lsrc, ldst, cnts)
        nums = out_n.reshape(_NPAD, 2 * _H)
        svals = out_s.reshape(_NPAD, 32)
        if i < 2:
            xlc, xrc = _tc_call(
                _mid_body, (_XO, _XO),
                nums, svals, Bo_m[i], Bo_v[i], Wl[i + 1], Wr[i + 1],
                Bl[i + 1], Br[i + 1])

    eps = jax.random.normal(jax.random.key(42), (_N, _H), dtype=jnp.float32)
    eps_pad = jnp.concatenate([eps, jnp.zeros((_NPAD - _N, _H), jnp.float32)])
    batch_pad = jnp.concatenate(
        [batch.astype(jnp.int32), jnp.full((_NPAD - _N,), 1 << 20, jnp.int32)]
    ).reshape(1, _NPAD)

    fc1_W, fc1_b, fc2_W, fc2_b = fc_params
    w2p = jnp.zeros((128, 128), jnp.float32).at[:, :2].set(fc2_W)
    b2p = jnp.zeros((1, 128), jnp.float32).at[0, :2].set(fc2_b)

    logp, mean_pad, logv_pad = _tc_call(
        _head_body,
        (jax.ShapeDtypeStruct((_G, 2), jnp.float32), _HO, _HO),
        nums, svals, Bo_m[2], Bo_v[2], eps_pad, batch_pad,
        fc1_W, fc1_b.reshape(1, 128), w2p, b2p)

    return (logp, mean_pad[:_N], logv_pad[:_N])


# speedup vs baseline: 1.1012x; 1.1012x over previous
import dataclasses

import jax
import jax.numpy as jnp
from jax import lax
from jax.experimental import pallas as pl
from jax.experimental.pallas import tpu as pltpu
from jax.experimental.pallas import tpu_sc as plsc

_N = 10000
_E = 320000
_DIN = 128
_H = 64
_G = 64
_NT = 32
_TPB = 320
_NPAD = _NT * _TPB
_E2 = _E + _N
_E2P = 330240
_SCK = 2560
_CK = 256
_CAPT = 24576
_CAPB = _CAPT + _SCK

_MESH = plsc.VectorSubcoreMesh(core_axis_name="c", subcore_axis_name="s")

_SC_PARAMS = pltpu.CompilerParams()
if "needs_layout_passes" in pltpu.CompilerParams.__dataclass_fields__:
    _SC_PARAMS = dataclasses.replace(_SC_PARAMS, needs_layout_passes=False)


def _wid():
    return lax.axis_index("s") * 2 + lax.axis_index("c")


def _prep_body(src_hbm, dst_hbm, lsrc_hbm, ldst_hbm, cnt_hbm,
               src_v, dst_v, stage_s, stage_d, cnt_v, sem_a, sem_b, sem):
    wid = _wid()
    lo = wid * _TPB

    zero16 = jnp.zeros((16,), jnp.int32)

    dump16 = jnp.full((16,), _TPB, jnp.int32)

    @pl.loop(0, _CAPB, step=16)
    def _(i):
        stage_s[pl.ds(i, 16)] = zero16
        stage_d[pl.ds(i, 16)] = dump16

    nchunks = _E2P // _SCK

    def issue(g, b):
        gc = jnp.minimum(g, nchunks - 1)
        s = sem_a if b == 0 else sem_b
        pltpu.async_copy(src_hbm.at[pl.ds(gc * _SCK, _SCK)], src_v.at[b], s)
        pltpu.async_copy(dst_hbm.at[pl.ds(gc * _SCK, _SCK)], dst_v.at[b], s)

    def wait(b):
        s = sem_a if b == 0 else sem_b
        pltpu.make_async_copy(src_hbm.at[pl.ds(0, _SCK)], src_v.at[b],
                              s).wait()
        pltpu.make_async_copy(dst_hbm.at[pl.ds(0, _SCK)], dst_v.at[b],
                              s).wait()

    def scan(b, cnt):
        def grp(i, cnt):
            s16 = src_v[b, pl.ds(i * 16, 16)]
            d16 = dst_v[b, pl.ds(i * 16, 16)]
            dl = d16 - lo
            m = (dl >= 0) & (dl < _TPB)
            plsc.store_compressed(stage_s.at[pl.ds(cnt, 16)], s16, mask=m)
            plsc.store_compressed(stage_d.at[pl.ds(cnt, 16)], dl, mask=m)
            pc = plsc.all_reduce_population_count(m)
            return cnt + lax.squeeze(lax.slice(pc, (0,), (1,)), (0,))

        return lax.cond(cnt <= _CAPT,
                        lambda c: lax.fori_loop(0, _SCK // 16, grp, c),
                        lambda c: c, cnt)

    issue(0, 0)
    issue(1, 1)

    def pair(p, cnt):
        g0 = 2 * p
        wait(0)
        cnt = scan(0, cnt)
        issue(g0 + 2, 0)
        wait(1)
        cnt = scan(1, cnt)
        issue(g0 + 3, 1)
        return cnt

    cnt = lax.fori_loop(0, (nchunks - 1) // 2, pair, jnp.int32(0))
    wait(0)
    cnt = scan(0, cnt)
    wait(1)
    cnt = jnp.minimum(cnt, jnp.int32(_CAPT))
    cnt_v[...] = jnp.full((16,), cnt, jnp.int32)
    pltpu.async_copy(cnt_v, cnt_hbm.at[pl.ds(wid * 16, 16)], sem).wait()
    pltpu.async_copy(stage_s.at[pl.ds(0, _CAPT)],
                     lsrc_hbm.at[pl.ds(wid * _CAPT, _CAPT)], sem).wait()
    pltpu.async_copy(stage_d.at[pl.ds(0, _CAPT)],
                     ldst_hbm.at[pl.ds(wid * _CAPT, _CAPT)], sem).wait()


def _prep(src, dst):
    return pl.kernel(
        _prep_body,
        out_type=(
            jax.ShapeDtypeStruct((_NT * _CAPT,), jnp.int32),
            jax.ShapeDtypeStruct((_NT * _CAPT,), jnp.int32),
            jax.ShapeDtypeStruct((_NT * 16,), jnp.int32),
        ),
        mesh=_MESH,
        scratch_types=[
            pltpu.VMEM((2, _SCK), jnp.int32),
            pltpu.VMEM((2, _SCK), jnp.int32),
            pltpu.VMEM((_CAPB,), jnp.int32),
            pltpu.VMEM((_CAPB,), jnp.int32),
            pltpu.VMEM((16,), jnp.int32),
            pltpu.SemaphoreType.DMA,
            pltpu.SemaphoreType.DMA,
            pltpu.SemaphoreType.DMA,
        ],
        compiler_params=_SC_PARAMS,
    )(src, dst)


def _edge_body(xlc_hbm, xrc_hbm, att_hbm, lsrc_hbm, ldst_hbm, cnt_hbm,
               outn_hbm, outs_hbm, xr_loc, att_v, cnt_v, idx_s0, idx_s1,
               idx_d0, idx_d1, rows, acc_n, acc_s,
               semi0, semi1, semg0, semg1, semm):
    wid = _wid()
    lo = wid * _TPB

    pltpu.async_copy(cnt_hbm.at[pl.ds(wid * 16, 16)], cnt_v, semm).wait()
    pltpu.async_copy(att_hbm, att_v, semm).wait()
    xr_cp = pltpu.async_copy(xrc_hbm.at[pl.ds(lo, _TPB)], xr_loc, semm)

    zero16 = jnp.zeros((16,), jnp.float32)

    @pl.loop(0, _TPB + 1)
    def _(r):
        for c in range(8):
            acc_n[r, pl.ds(c * 16, 16)] = zero16

    @pl.loop(0, 32 * (_TPB + 1), step=16)
    def _(i):
        acc_s[pl.ds(i, 16)] = zero16

    xr_cp.wait()

    cnt = cnt_v[...][0]
    a = [att_v[pl.ds(c * 16, 16)] for c in range(8)]
    lane0f = jnp.where(lax.iota(jnp.int32, 16) == 0, 1.0, 0.0)

    nch = (cnt + _CK - 1) // _CK
    gmax = _CAPT // _CK - 1

    def issue_idx(g, b):
        base = wid * _CAPT + jnp.minimum(g, gmax) * _CK
        s = semi0 if b == 0 else semi1
        i_s = idx_s0 if b == 0 else idx_s1
        i_d = idx_d0 if b == 0 else idx_d1
        pltpu.async_copy(lsrc_hbm.at[pl.ds(base, _CK)], i_s, s)
        pltpu.async_copy(ldst_hbm.at[pl.ds(base, _CK)], i_d, s)

    def wait_idx(b):
        s = semi0 if b == 0 else semi1
        i_s = idx_s0 if b == 0 else idx_s1
        i_d = idx_d0 if b == 0 else idx_d1
        pltpu.make_async_copy(lsrc_hbm.at[pl.ds(0, _CK)], i_s, s).wait()
        pltpu.make_async_copy(ldst_hbm.at[pl.ds(0, _CK)], i_d, s).wait()

    def issue_gather(b):
        i_s = idx_s0 if b == 0 else idx_s1
        pltpu.async_copy(xlc_hbm.at[i_s], rows, semg0)

    def wait_gather(b):
        i_s = idx_s0 if b == 0 else idx_s1
        pltpu.make_async_copy(xlc_hbm.at[i_s], rows, semg0).wait()

    last = jnp.full((16,), 15, jnp.int32)

    def compute(g, b):
        i_d = idx_d0 if b == 0 else idx_d1

        def grp(g16, _):
            dl16 = i_d[pl.ds(g16 * 16, 16)]
            for j in range(16):
                e = g16 * 16 + j
                dl = dl16[j]
                xm = [rows[e, pl.ds(c * 16, 16)] for c in range(4)]
                xv = [rows[e, pl.ds(64 + c * 16, 16)] for c in range(4)]
                pm = pv = None
                for c in range(4):
                    tm = xm[c] + xr_loc[dl, pl.ds(c * 16, 16)]
                    tm = jnp.maximum(tm, 0.2 * tm) * a[c]
                    pm = tm if pm is None else pm + tm
                    tv = xv[c] + xr_loc[dl, pl.ds(64 + c * 16, 16)]
                    tv = jnp.maximum(tv, 0.2 * tv) * a[4 + c]
                    pv = tv if pv is None else pv + tv
                evm = jnp.exp(plsc.cumsum(pm).at[last]
                              .get(mode="promise_in_bounds"))
                evv = jnp.exp(plsc.cumsum(pv).at[last]
                              .get(mode="promise_in_bounds"))
                for c in range(4):
                    plsc.addupdate(acc_n.at[dl, pl.ds(c * 16, 16)],
                                   evm * xm[c])
                    plsc.addupdate(acc_n.at[dl, pl.ds(64 + c * 16, 16)],
                                   evv * xv[c])
                plsc.addupdate(acc_s.at[pl.ds(dl * 32, 16)], evm * lane0f)
                plsc.addupdate(acc_s.at[pl.ds(dl * 32 + 16, 16)],
                               evv * lane0f)
            return 0

        lax.fori_loop(0, _CK // 16, grp, 0)

    def chunk(g, _):
        issue_idx(g, 0)
        wait_idx(0)
        issue_gather(0)
        wait_gather(0)
        compute(g, 0)
        return 0

    lax.fori_loop(0, nch, chunk, 0)

    pltpu.async_copy(acc_n.at[pl.ds(0, _TPB)], outn_hbm.at[wid],
                     semm).wait()
    pltpu.async_copy(acc_s.at[pl.ds(0, 32 * _TPB)],
                     outs_hbm.at[pl.ds(wid * 32 * _TPB, 32 * _TPB)],
                     semm).wait()


def _edge_call(xlc, xrc, att, lsrc, ldst, cnts):
    return pl.kernel(
        _edge_body,
        out_type=(
            jax.ShapeDtypeStruct((_NT, _TPB, 2 * _H), jnp.float32),
            jax.ShapeDtypeStruct((_NT * _TPB * 32,), jnp.float32),
        ),
        mesh=_MESH,
        scratch_types=[
            pltpu.VMEM((_TPB, 2 * _H), jnp.float32),
            pltpu.VMEM((2 * _H,), jnp.float32),
            pltpu.VMEM((16,), jnp.int32),
            pltpu.VMEM((_CK,), jnp.int32),
            pltpu.VMEM((_CK,), jnp.int32),
            pltpu.VMEM((_CK,), jnp.int32),
            pltpu.VMEM((_CK,), jnp.int32),
            pltpu.VMEM((_CK, 2 * _H), jnp.float32),
            pltpu.VMEM((_TPB + 1, 2 * _H), jnp.float32),
            pltpu.VMEM((32 * (_TPB + 1),), jnp.float32),
            pltpu.SemaphoreType.DMA,
            pltpu.SemaphoreType.DMA,
            pltpu.SemaphoreType.DMA,
            pltpu.SemaphoreType.DMA,
            pltpu.SemaphoreType.DMA,
        ],
        compiler_params=_SC_PARAMS,
    )(xlc, xrc, att, lsrc, ldst, cnts)


def _elu(h):
    return jnp.where(h > 0, h, jnp.exp(h) - 1.0)


def _finish(num, s, b):
    return _elu(num / (s + 1e-16) + b)


def _proj_first_body(x_ref, wl_ref, wr_ref, bl_ref, br_ref, xlc, xrc):
    x = x_ref[...]
    xlc[...] = jnp.dot(x, wl_ref[...],
                       preferred_element_type=jnp.float32) + bl_ref[...]
    xrc[...] = jnp.dot(x, wr_ref[...],
                       preferred_element_type=jnp.float32) + br_ref[...]


def _mid_body(nums_ref, sv_ref, bpm_ref, bpv_ref, wl_ref, wr_ref,
              bl_ref, br_ref, xlc, xrc):
    nums = nums_ref[...]
    sv = sv_ref[...]
    h = jnp.concatenate(
        [_finish(nums[:, :_H], sv[:, 0:1], bpm_ref[...]),
         _finish(nums[:, _H:], sv[:, 16:17], bpv_ref[...])], axis=1)
    xlc[...] = jnp.dot(h, wl_ref[...],
                       preferred_element_type=jnp.float32) + bl_ref[...]
    xrc[...] = jnp.dot(h, wr_ref[...],
                       preferred_element_type=jnp.float32) + br_ref[...]


def _head_body(nums_ref, sv_ref, bpm_ref, bpv_ref, eps_ref, batch_ref,
               w1_ref, b1_ref, w2_ref, b2_ref, logp, mean_o, logv_o):
    nums = nums_ref[...]
    sv = sv_ref[...]
    mean = _finish(nums[:, :_H], sv[:, 0:1], bpm_ref[...])
    logv = _finish(nums[:, _H:], sv[:, 16:17], bpv_ref[...])
    mean_o[...] = mean
    logv_o[...] = logv
    z = mean + eps_ref[...] * jnp.exp(0.5 * logv)
    gi = lax.broadcasted_iota(jnp.int32, (_G, _NPAD), 0)
    onehot = jnp.where(batch_ref[...] == gi, 1.0, 0.0)
    sums = jnp.dot(onehot, z, preferred_element_type=jnp.float32)
    cnt = jnp.sum(onehot, axis=1, keepdims=True)
    zg = sums / jnp.maximum(cnt, 1.0)
    h = jnp.maximum(
        jnp.dot(zg, w1_ref[...], preferred_element_type=jnp.float32)
        + b1_ref[...], 0.0)
    out = jnp.dot(h, w2_ref[...], preferred_element_type=jnp.float32) \
        + b2_ref[...]
    logits = out[:, :2]
    m = jnp.max(logits, axis=1, keepdims=True)
    lse = m + jnp.log(jnp.sum(jnp.exp(logits - m), axis=1, keepdims=True))
    logp[...] = logits - lse


def _tc_call(body, out_type, *args):
    return pl.pallas_call(body, out_shape=out_type)(*args)


_XO = jax.ShapeDtypeStruct((_NPAD, 2 * _H), jnp.float32)
_HO = jax.ShapeDtypeStruct((_NPAD, _H), jnp.float32)


def _blockdiag(wm, wv):
    z = jnp.zeros((2 * _H, 2 * _H), jnp.float32)
    return z.at[:_H, :_H].set(wm).at[_H:, _H:].set(wv)


def kernel(x, edge_index, batch, mean_params, var_params, fc_params):
    ar = jnp.arange(_N, dtype=jnp.int32)
    npad_e = _E2P - _E2
    src = jnp.concatenate([edge_index[0].astype(jnp.int32), ar,
                           jnp.zeros((npad_e,), jnp.int32)])
    dst = jnp.concatenate([edge_index[1].astype(jnp.int32), ar,
                           jnp.full((npad_e,), 1 << 30, jnp.int32)])
    lsrc, ldst, cnts = _prep(src, dst)

    x_pad = jnp.concatenate([x, jnp.zeros((_NPAD - _N, _DIN), x.dtype)])

    Wl, Wr, Bl, Br, Att, Bo_m, Bo_v = [], [], [], [], [], [], []
    for i in range(3):
        wlm, blm, wrm, brm, attm, bm = mean_params[i]
        wlv, blv, wrv, brv, attv, bv = var_params[i]
        if i == 0:
            Wl.append(jnp.concatenate([wlm, wlv], axis=1))
            Wr.append(jnp.concatenate([wrm, wrv], axis=1))
        else:
            Wl.append(_blockdiag(wlm, wlv))
            Wr.append(_blockdiag(wrm, wrv))
        Bl.append(jnp.concatenate([blm, blv]).reshape(1, 2 * _H))
        Br.append(jnp.concatenate([brm, brv]).reshape(1, 2 * _H))
        Att.append(jnp.concatenate([attm, attv]))
        Bo_m.append(bm.reshape(1, _H))
        Bo_v.append(bv.reshape(1, _H))

    xlc, xrc = _tc_call(_proj_first_body, (_XO, _XO),
                        x_pad, Wl[0], Wr[0], Bl[0], Br[0])

    for i in range(3):
        out_n, out_s = _edge_call(xlc, xrc, Att[i], lsrc, ldst, cnts)
        nums = out_n.reshape(_NPAD, 2 * _H)
        svals = out_s.reshape(_NPAD, 32)
        if i < 2:
            xlc, xrc = _tc_call(
                _mid_body, (_XO, _XO),
                nums, svals, Bo_m[i], Bo_v[i], Wl[i + 1], Wr[i + 1],
                Bl[i + 1], Br[i + 1])

    eps = jax.random.normal(jax.random.key(42), (_N, _H), dtype=jnp.float32)
    eps_pad = jnp.concatenate([eps, jnp.zeros((_NPAD - _N, _H), jnp.float32)])
    batch_pad = jnp.concatenate(
        [batch.astype(jnp.int32), jnp.full((_NPAD - _N,), 1 << 20, jnp.int32)]
    ).reshape(1, _NPAD)

    fc1_W, fc1_b, fc2_W, fc2_b = fc_params
    w2p = jnp.zeros((128, 128), jnp.float32).at[:, :2].set(fc2_W)
    b2p = jnp.zeros((1, 128), jnp.float32).at[0, :2].set(fc2_b)

    logp, mean_pad, logv_pad = _tc_call(
        _head_body,
        (jax.ShapeDtypeStruct((_G, 2), jnp.float32), _HO, _HO),
        nums, svals, Bo_m[2], Bo_v[2], eps_pad, batch_pad,
        fc1_W, fc1_b.reshape(1, 128), w2p, b2p)

    return (logp, mean_pad[:_N], logv_pad[:_N])


# speedup vs baseline: 1.1065x; 1.0048x over previous
import dataclasses

import jax
import jax.numpy as jnp
from jax import lax
from jax.experimental import pallas as pl
from jax.experimental.pallas import tpu as pltpu
from jax.experimental.pallas import tpu_sc as plsc

_N = 10000
_E = 320000
_DIN = 128
_H = 64
_G = 64
_NT = 32
_TPB = 320
_NPAD = _NT * _TPB
_E2 = _E + _N
_E2P = 330240
_SCK = 2560
_CK = 128
_CAPT = 24576
_CAPB = _CAPT + _SCK

_MESH = plsc.VectorSubcoreMesh(core_axis_name="c", subcore_axis_name="s")

_SC_PARAMS = pltpu.CompilerParams()
if "needs_layout_passes" in pltpu.CompilerParams.__dataclass_fields__:
    _SC_PARAMS = dataclasses.replace(_SC_PARAMS, needs_layout_passes=False)


def _wid():
    return lax.axis_index("s") * 2 + lax.axis_index("c")


def _prep_body(src_hbm, dst_hbm, lsrc_hbm, ldst_hbm, cnt_hbm,
               src_v, dst_v, stage_s, stage_d, cnt_v, sem_a, sem_b, sem):
    wid = _wid()
    lo = wid * _TPB

    zero16 = jnp.zeros((16,), jnp.int32)

    dump16 = jnp.full((16,), _TPB, jnp.int32)

    @pl.loop(0, _CAPB, step=16)
    def _(i):
        stage_s[pl.ds(i, 16)] = zero16
        stage_d[pl.ds(i, 16)] = dump16

    nchunks = _E2P // _SCK

    def issue(g, b):
        gc = jnp.minimum(g, nchunks - 1)
        s = sem_a if b == 0 else sem_b
        pltpu.async_copy(src_hbm.at[pl.ds(gc * _SCK, _SCK)], src_v.at[b], s)
        pltpu.async_copy(dst_hbm.at[pl.ds(gc * _SCK, _SCK)], dst_v.at[b], s)

    def wait(b):
        s = sem_a if b == 0 else sem_b
        pltpu.make_async_copy(src_hbm.at[pl.ds(0, _SCK)], src_v.at[b],
                              s).wait()
        pltpu.make_async_copy(dst_hbm.at[pl.ds(0, _SCK)], dst_v.at[b],
                              s).wait()

    def scan(b, cnt):
        def grp(i, cnt):
            s16 = src_v[b, pl.ds(i * 16, 16)]
            d16 = dst_v[b, pl.ds(i * 16, 16)]
            dl = d16 - lo
            m = (dl >= 0) & (dl < _TPB)
            plsc.store_compressed(stage_s.at[pl.ds(cnt, 16)], s16, mask=m)
            plsc.store_compressed(stage_d.at[pl.ds(cnt, 16)], dl, mask=m)
            pc = plsc.all_reduce_population_count(m)
            return cnt + lax.squeeze(lax.slice(pc, (0,), (1,)), (0,))

        return lax.cond(cnt <= _CAPT,
                        lambda c: lax.fori_loop(0, _SCK // 16, grp, c),
                        lambda c: c, cnt)

    issue(0, 0)
    issue(1, 1)

    def pair(p, cnt):
        g0 = 2 * p
        wait(0)
        cnt = scan(0, cnt)
        issue(g0 + 2, 0)
        wait(1)
        cnt = scan(1, cnt)
        issue(g0 + 3, 1)
        return cnt

    cnt = lax.fori_loop(0, (nchunks - 1) // 2, pair, jnp.int32(0))
    wait(0)
    cnt = scan(0, cnt)
    wait(1)
    cnt = jnp.minimum(cnt, jnp.int32(_CAPT))
    cnt_v[...] = jnp.full((16,), cnt, jnp.int32)
    pltpu.async_copy(cnt_v, cnt_hbm.at[pl.ds(wid * 16, 16)], sem).wait()
    pltpu.async_copy(stage_s.at[pl.ds(0, _CAPT)],
                     lsrc_hbm.at[pl.ds(wid * _CAPT, _CAPT)], sem).wait()
    pltpu.async_copy(stage_d.at[pl.ds(0, _CAPT)],
                     ldst_hbm.at[pl.ds(wid * _CAPT, _CAPT)], sem).wait()


def _prep(src, dst):
    return pl.kernel(
        _prep_body,
        out_type=(
            jax.ShapeDtypeStruct((_NT * _CAPT,), jnp.int32),
            jax.ShapeDtypeStruct((_NT * _CAPT,), jnp.int32),
            jax.ShapeDtypeStruct((_NT * 16,), jnp.int32),
        ),
        mesh=_MESH,
        scratch_types=[
            pltpu.VMEM((2, _SCK), jnp.int32),
            pltpu.VMEM((2, _SCK), jnp.int32),
            pltpu.VMEM((_CAPB,), jnp.int32),
            pltpu.VMEM((_CAPB,), jnp.int32),
            pltpu.VMEM((16,), jnp.int32),
            pltpu.SemaphoreType.DMA,
            pltpu.SemaphoreType.DMA,
            pltpu.SemaphoreType.DMA,
        ],
        compiler_params=_SC_PARAMS,
    )(src, dst)


def _edge_body(xlc_hbm, xrc_hbm, att_hbm, lsrc_hbm, ldst_hbm, cnt_hbm,
               outn_hbm, outs_hbm, xr_loc, att_v, cnt_v, idx_s, idx_d,
               rows, acc_n, acc_s, semi, semg, semm):
    wid = _wid()
    lo = wid * _TPB

    pltpu.async_copy(cnt_hbm.at[pl.ds(wid * 16, 16)], cnt_v, semm).wait()
    pltpu.async_copy(att_hbm, att_v, semm).wait()
    xr_cp = pltpu.async_copy(xrc_hbm.at[pl.ds(lo, _TPB)], xr_loc, semm)

    zero16 = jnp.zeros((16,), jnp.float32)

    @pl.loop(0, _TPB + 1)
    def _(r):
        for c in range(8):
            acc_n[r, pl.ds(c * 16, 16)] = zero16

    @pl.loop(0, 32 * (_TPB + 1), step=16)
    def _(i):
        acc_s[pl.ds(i, 16)] = zero16

    xr_cp.wait()

    cnt = cnt_v[...][0]
    a = [att_v[pl.ds(c * 16, 16)] for c in range(8)]
    lane0f = jnp.where(lax.iota(jnp.int32, 16) == 0, 1.0, 0.0)

    nch = (cnt + _CK - 1) // _CK
    gmax = _CAPT // _CK - 1

    def half(g):
        return (g % 2) * _CK

    def issue_idx(g):
        base = wid * _CAPT + jnp.minimum(g, gmax) * _CK
        off = half(g)
        pltpu.async_copy(lsrc_hbm.at[pl.ds(base, _CK)],
                         idx_s.at[pl.ds(off, _CK)], semi)
        pltpu.async_copy(ldst_hbm.at[pl.ds(base, _CK)],
                         idx_d.at[pl.ds(off, _CK)], semi)

    def wait_idx():
        pltpu.make_async_copy(lsrc_hbm.at[pl.ds(0, _CK)],
                              idx_s.at[pl.ds(0, _CK)], semi).wait()
        pltpu.make_async_copy(ldst_hbm.at[pl.ds(0, _CK)],
                              idx_d.at[pl.ds(0, _CK)], semi).wait()

    def issue_gather(g):
        off = half(g)
        pltpu.async_copy(xlc_hbm.at[idx_s.at[pl.ds(off, _CK)]],
                         rows.at[pl.ds(off, _CK)], semg)

    def wait_gather():
        pltpu.make_async_copy(xlc_hbm.at[idx_s.at[pl.ds(0, _CK)]],
                              rows.at[pl.ds(0, _CK)], semg).wait()

    last = jnp.full((16,), 15, jnp.int32)

    def compute(g):
        off = half(g)

        def grp(g16, _):
            dl16 = idx_d[pl.ds(off + g16 * 16, 16)]
            for j in range(16):
                e = g16 * 16 + j
                dl = dl16[j]
                xm = [rows[off + e, pl.ds(c * 16, 16)] for c in range(4)]
                xv = [rows[off + e, pl.ds(64 + c * 16, 16)]
                      for c in range(4)]
                pm = pv = None
                for c in range(4):
                    tm = xm[c] + xr_loc[dl, pl.ds(c * 16, 16)]
                    tm = jnp.maximum(tm, 0.2 * tm) * a[c]
                    pm = tm if pm is None else pm + tm
                    tv = xv[c] + xr_loc[dl, pl.ds(64 + c * 16, 16)]
                    tv = jnp.maximum(tv, 0.2 * tv) * a[4 + c]
                    pv = tv if pv is None else pv + tv
                evm = jnp.exp(plsc.cumsum(pm).at[last]
                              .get(mode="promise_in_bounds"))
                evv = jnp.exp(plsc.cumsum(pv).at[last]
                              .get(mode="promise_in_bounds"))
                for c in range(4):
                    plsc.addupdate(acc_n.at[dl, pl.ds(c * 16, 16)],
                                   evm * xm[c])
                    plsc.addupdate(acc_n.at[dl, pl.ds(64 + c * 16, 16)],
                                   evv * xv[c])
                plsc.addupdate(acc_s.at[pl.ds(dl * 32, 16)], evm * lane0f)
                plsc.addupdate(acc_s.at[pl.ds(dl * 32 + 16, 16)],
                               evv * lane0f)
            return 0

        lax.fori_loop(0, _CK // 16, grp, 0)

    issue_idx(0)
    wait_idx()
    issue_gather(0)
    issue_idx(1)

    def chunk(g, _):
        wait_idx()
        issue_gather(g + 1)
        wait_gather()
        compute(g)
        issue_idx(g + 2)
        return 0

    lax.fori_loop(0, nch, chunk, 0)
    wait_gather()
    wait_idx()

    pltpu.async_copy(acc_n.at[pl.ds(0, _TPB)], outn_hbm.at[wid],
                     semm).wait()
    pltpu.async_copy(acc_s.at[pl.ds(0, 32 * _TPB)],
                     outs_hbm.at[pl.ds(wid * 32 * _TPB, 32 * _TPB)],
                     semm).wait()


def _edge_call(xlc, xrc, att, lsrc, ldst, cnts):
    return pl.kernel(
        _edge_body,
        out_type=(
            jax.ShapeDtypeStruct((_NT, _TPB, 2 * _H), jnp.float32),
            jax.ShapeDtypeStruct((_NT * _TPB * 32,), jnp.float32),
        ),
        mesh=_MESH,
        scratch_types=[
            pltpu.VMEM((_TPB, 2 * _H), jnp.float32),
            pltpu.VMEM((2 * _H,), jnp.float32),
            pltpu.VMEM((16,), jnp.int32),
            pltpu.VMEM((2 * _CK,), jnp.int32),
            pltpu.VMEM((2 * _CK,), jnp.int32),
            pltpu.VMEM((2 * _CK, 2 * _H), jnp.float32),
            pltpu.VMEM((_TPB + 1, 2 * _H), jnp.float32),
            pltpu.VMEM((32 * (_TPB + 1),), jnp.float32),
            pltpu.SemaphoreType.DMA,
            pltpu.SemaphoreType.DMA,
            pltpu.SemaphoreType.DMA,
        ],
        compiler_params=_SC_PARAMS,
    )(xlc, xrc, att, lsrc, ldst, cnts)


def _elu(h):
    return jnp.where(h > 0, h, jnp.exp(h) - 1.0)


def _finish(num, s, b):
    return _elu(num / (s + 1e-16) + b)


def _proj_first_body(x_ref, wl_ref, wr_ref, bl_ref, br_ref, xlc, xrc):
    x = x_ref[...]
    xlc[...] = jnp.dot(x, wl_ref[...],
                       preferred_element_type=jnp.float32) + bl_ref[...]
    xrc[...] = jnp.dot(x, wr_ref[...],
                       preferred_element_type=jnp.float32) + br_ref[...]


def _mid_body(nums_ref, sv_ref, bpm_ref, bpv_ref, wl_ref, wr_ref,
              bl_ref, br_ref, xlc, xrc):
    nums = nums_ref[...]
    sv = sv_ref[...]
    h = jnp.concatenate(
        [_finish(nums[:, :_H], sv[:, 0:1], bpm_ref[...]),
         _finish(nums[:, _H:], sv[:, 16:17], bpv_ref[...])], axis=1)
    xlc[...] = jnp.dot(h, wl_ref[...],
                       preferred_element_type=jnp.float32) + bl_ref[...]
    xrc[...] = jnp.dot(h, wr_ref[...],
                       preferred_element_type=jnp.float32) + br_ref[...]


def _head_body(nums_ref, sv_ref, bpm_ref, bpv_ref, eps_ref, batch_ref,
               w1_ref, b1_ref, w2_ref, b2_ref, logp, mean_o, logv_o):
    nums = nums_ref[...]
    sv = sv_ref[...]
    mean = _finish(nums[:, :_H], sv[:, 0:1], bpm_ref[...])
    logv = _finish(nums[:, _H:], sv[:, 16:17], bpv_ref[...])
    mean_o[...] = mean
    logv_o[...] = logv
    z = mean + eps_ref[...] * jnp.exp(0.5 * logv)
    gi = lax.broadcasted_iota(jnp.int32, (_G, _NPAD), 0)
    onehot = jnp.where(batch_ref[...] == gi, 1.0, 0.0)
    sums = jnp.dot(onehot, z, preferred_element_type=jnp.float32)
    cnt = jnp.sum(onehot, axis=1, keepdims=True)
    zg = sums / jnp.maximum(cnt, 1.0)
    h = jnp.maximum(
        jnp.dot(zg, w1_ref[...], preferred_element_type=jnp.float32)
        + b1_ref[...], 0.0)
    out = jnp.dot(h, w2_ref[...], preferred_element_type=jnp.float32) \
        + b2_ref[...]
    logits = out[:, :2]
    m = jnp.max(logits, axis=1, keepdims=True)
    lse = m + jnp.log(jnp.sum(jnp.exp(logits - m), axis=1, keepdims=True))
    logp[...] = logits - lse


def _tc_call(body, out_type, *args):
    return pl.pallas_call(body, out_shape=out_type)(*args)


_XO = jax.ShapeDtypeStruct((_NPAD, 2 * _H), jnp.float32)
_HO = jax.ShapeDtypeStruct((_NPAD, _H), jnp.float32)


def _blockdiag(wm, wv):
    z = jnp.zeros((2 * _H, 2 * _H), jnp.float32)
    return z.at[:_H, :_H].set(wm).at[_H:, _H:].set(wv)


def kernel(x, edge_index, batch, mean_params, var_params, fc_params):
    ar = jnp.arange(_N, dtype=jnp.int32)
    npad_e = _E2P - _E2
    src = jnp.concatenate([edge_index[0].astype(jnp.int32), ar,
                           jnp.zeros((npad_e,), jnp.int32)])
    dst = jnp.concatenate([edge_index[1].astype(jnp.int32), ar,
                           jnp.full((npad_e,), 1 << 30, jnp.int32)])
    lsrc, ldst, cnts = _prep(src, dst)

    x_pad = jnp.concatenate([x, jnp.zeros((_NPAD - _N, _DIN), x.dtype)])

    Wl, Wr, Bl, Br, Att, Bo_m, Bo_v = [], [], [], [], [], [], []
    for i in range(3):
        wlm, blm, wrm, brm, attm, bm = mean_params[i]
        wlv, blv, wrv, brv, attv, bv = var_params[i]
        if i == 0:
            Wl.append(jnp.concatenate([wlm, wlv], axis=1))
            Wr.append(jnp.concatenate([wrm, wrv], axis=1))
        else:
            Wl.append(_blockdiag(wlm, wlv))
            Wr.append(_blockdiag(wrm, wrv))
        Bl.append(jnp.concatenate([blm, blv]).reshape(1, 2 * _H))
        Br.append(jnp.concatenate([brm, brv]).reshape(1, 2 * _H))
        Att.append(jnp.concatenate([attm, attv]))
        Bo_m.append(bm.reshape(1, _H))
        Bo_v.append(bv.reshape(1, _H))

    xlc, xrc = _tc_call(_proj_first_body, (_XO, _XO),
                        x_pad, Wl[0], Wr[0], Bl[0], Br[0])

    for i in range(3):
        out_n, out_s = _edge_call(xlc, xrc, Att[i], lsrc, ldst, cnts)
        nums = out_n.reshape(_NPAD, 2 * _H)
        svals = out_s.reshape(_NPAD, 32)
        if i < 2:
            xlc, xrc = _tc_call(
                _mid_body, (_XO, _XO),
                nums, svals, Bo_m[i], Bo_v[i], Wl[i + 1], Wr[i + 1],
                Bl[i + 1], Br[i + 1])

    eps = jax.random.normal(jax.random.key(42), (_N, _H), dtype=jnp.float32)
    eps_pad = jnp.concatenate([eps, jnp.zeros((_NPAD - _N, _H), jnp.float32)])
    batch_pad = jnp.concatenate(
        [batch.astype(jnp.int32), jnp.full((_NPAD - _N,), 1 << 20, jnp.int32)]
    ).reshape(1, _NPAD)

    fc1_W, fc1_b, fc2_W, fc2_b = fc_params
    w2p = jnp.zeros((128, 128), jnp.float32).at[:, :2].set(fc2_W)
    b2p = jnp.zeros((1, 128), jnp.float32).at[0, :2].set(fc2_b)

    logp, mean_pad, logv_pad = _tc_call(
        _head_body,
        (jax.ShapeDtypeStruct((_G, 2), jnp.float32), _HO, _HO),
        nums, svals, Bo_m[2], Bo_v[2], eps_pad, batch_pad,
        fc1_W, fc1_b.reshape(1, 128), w2p, b2p)

    return (logp, mean_pad[:_N], logv_pad[:_N])
